# baseline (speedup 1.0000x reference)
import functools

import jax
import jax.numpy as jnp
from jax import lax
from jax.experimental import pallas as pl

_B, _Q, _C1 = 32, 1000, 4
_T, _P = 100, 50
_PAIRS = _B * _T
_NQ = _B * _Q
_GRID = 5
_PT = _PAIRS // _GRID
_QT = _NQ // _GRID
_W = 128
_NW = _PT // _W


def _loss_kernel(logits_ref, labels_ref, s_ref, t_ref, se_ref, te_ref, out_ref):
    g = pl.program_id(0)

    @pl.when(g == 0)
    def _init():
        out_ref[...] = jnp.zeros_like(out_ref)

    lg = logits_ref[...]
    m = jnp.max(lg, axis=0, keepdims=True)
    lse = jnp.log(jnp.sum(jnp.exp(lg - m), axis=0, keepdims=True)) + m
    lab = labels_ref[...]
    cls = lax.broadcasted_iota(jnp.int32, (_C1, _QT), 0)
    matched = jnp.sum(jnp.where(cls == lab, lg, 0.0), axis=0, keepdims=True)
    ce = jnp.sum(lse - matched) / _NQ

    poly = 0.0
    for w in range(_NW):
        sl = slice(w * _W, (w + 1) * _W)
        sx = s_ref[0, :, sl]
        sy = s_ref[1, :, sl]
        macc = None
        acc1 = None
        for j in range(_P):
            txj = t_ref[0, j:j + 1, sl]
            tyj = t_ref[1, j:j + 1, sl]
            d = jnp.abs(sx - txj) + jnp.abs(sy - tyj)
            macc = d if macc is None else jnp.minimum(macc, d)
            cmin = jnp.min(d, axis=0, keepdims=True)
            c32 = cmin.astype(jnp.float32)
            acc1 = c32 if acc1 is None else acc1 + c32
        per_t = acc1 + jnp.sum(macc.astype(jnp.float32), axis=0, keepdims=True)
        poly = poly + jnp.sum(per_t)
    poly = poly * (0.5 / (_PAIRS * _P))

    sdx = se_ref[0, 1, :] - se_ref[0, 0, :]
    sdy = se_ref[1, 1, :] - se_ref[1, 0, :]
    tdx = te_ref[0, 1, :] - te_ref[0, 0, :]
    tdy = te_ref[1, 1, :] - te_ref[1, 0, :]
    sn = jnp.sqrt(sdx * sdx + sdy * sdy) + 1e-6
    tn = jnp.sqrt(tdx * tdx + tdy * tdy) + 1e-6
    cos = (sdx * tdx + sdy * tdy) / (sn * tn)
    direc = jnp.sum(1.0 - cos) / _PAIRS

    idx = lax.broadcasted_iota(jnp.int32, (3,), 0)
    contrib = (jnp.where(idx == 0, ce, 0.0)
               + jnp.where(idx == 1, poly, 0.0)
               + jnp.where(idx == 2, direc, 0.0))
    out_ref[...] = out_ref[...] + contrib


@jax.jit
def kernel(pred_logits, pred_polylines, tgt_labels, tgt_polylines):
    B, Q, C1 = pred_logits.shape
    T = tgt_labels.shape[1]
    P = pred_polylines.shape[2]

    logits_t = jnp.transpose(pred_logits.reshape(B * Q, C1), (1, 0))
    labels_full = jnp.concatenate(
        [tgt_labels.astype(jnp.int32),
         jnp.full((B, Q - T), C1 - 1, dtype=jnp.int32)], axis=1)
    labels_full = labels_full.reshape(1, B * Q)
    src = pred_polylines[:, :T]
    s_t = jnp.transpose(src.astype(jnp.bfloat16), (3, 2, 0, 1)).reshape(2, P, B * T)
    t_t = jnp.transpose(tgt_polylines.astype(jnp.bfloat16), (3, 2, 0, 1)).reshape(2, P, B * T)
    s_e = jnp.transpose(src[:, :, ::P - 1], (3, 2, 0, 1)).reshape(2, 2, B * T)
    t_e = jnp.transpose(tgt_polylines[:, :, ::P - 1], (3, 2, 0, 1)).reshape(2, 2, B * T)

    out = pl.pallas_call(
        _loss_kernel,
        grid=(_GRID,),
        in_specs=[
            pl.BlockSpec((C1, _QT), lambda g: (0, g)),
            pl.BlockSpec((1, _QT), lambda g: (0, g)),
            pl.BlockSpec((2, P, _PT), lambda g: (0, 0, g)),
            pl.BlockSpec((2, P, _PT), lambda g: (0, 0, g)),
            pl.BlockSpec((2, 2, _PT), lambda g: (0, 0, g)),
            pl.BlockSpec((2, 2, _PT), lambda g: (0, 0, g)),
        ],
        out_specs=pl.BlockSpec((3,), lambda g: (0,)),
        out_shape=jax.ShapeDtypeStruct((3,), jnp.float32),
    )(logits_t, labels_full, s_t, t_t, s_e, t_e)
    return out
